# baseline (speedup 1.0000x reference)
import functools

import jax
import jax.numpy as jnp
from jax import lax
from jax.experimental import pallas as pl
from jax.experimental.pallas import tpu as pltpu
from jax.experimental.pallas import tpu_sc as plsc

B = 16384
F = 26
V = 100000
D = 16

NC = 2
NS = 16
NW = NC * NS
SPT = B // NW
C = 128
NCHUNK = SPT // C


def _fm_body(idx_hbm, emb_hbm, lin_hbm, linb_hbm, out_hbm,
             idx_v, rows_v, lin_v, out_v, linb_v, sem):
    cid = lax.axis_index("c")
    sid = lax.axis_index("s")
    wid = sid * NC + cid

    pltpu.sync_copy(linb_hbm, linb_v)
    zeros16 = jnp.zeros((16,), jnp.int32)
    iota16 = lax.iota(jnp.int32, 16)
    bvec = linb_v[...]

    def chunk_body(c, carry):
        base = wid * SPT + c * C
        ibase = base * F
        pltpu.sync_copy(idx_hbm.at[pl.ds(ibase, C * F)], idx_v)
        pltpu.async_copy(emb_hbm.at[idx_v], rows_v, sem).wait()
        pltpu.async_copy(lin_hbm.at[idx_v], lin_v, sem).wait()

        def group_body(g, carry2):
            rbase = g * (16 * F) + iota16 * F
            acc = bvec
            lin_tot = plsc.load_gather(lin_v, [rbase])
            for f in range(1, F):
                lin_tot = lin_tot + plsc.load_gather(lin_v, [rbase + f])
            acc = acc + lin_tot
            for d in range(D):
                dsplat = jnp.full((16,), d, jnp.int32)
                s = plsc.load_gather(rows_v, [rbase, dsplat])
                ss = s * s
                for f in range(1, F):
                    v = plsc.load_gather(rows_v, [rbase + f, dsplat])
                    s = s + v
                    ss = ss + v * v
                acc = acc + 0.5 * (s * s - ss)
            out_v[pl.ds(g * 16, 16)] = 1.0 / (1.0 + jnp.exp(-acc))
            return carry2

        lax.fori_loop(0, C // 16, group_body, 0)
        pltpu.sync_copy(out_v, out_hbm.at[pl.ds(base, C)])
        return carry

    lax.fori_loop(0, NCHUNK, chunk_body, 0)


@functools.cache
def _fm_sc():
    return pl.kernel(
        _fm_body,
        out_type=jax.ShapeDtypeStruct((B,), jnp.float32),
        mesh=plsc.VectorSubcoreMesh(
            core_axis_name="c", subcore_axis_name="s",
            num_cores=NC, num_subcores=NS),
        compiler_params=pltpu.CompilerParams(
            needs_layout_passes=False, use_tc_tiling_on_sc=False),
        scratch_types=[
            pltpu.VMEM((C * F,), jnp.int32),
            pltpu.VMEM((C * F, D), jnp.float32),
            pltpu.VMEM((C * F,), jnp.float32),
            pltpu.VMEM((C,), jnp.float32),
            pltpu.VMEM((16,), jnp.float32),
            pltpu.SemaphoreType.DMA,
        ],
    )


def kernel(indices, emb_table, lin_w, lin_b):
    flat_idx = (indices.astype(jnp.int32)
                + (jnp.arange(F, dtype=jnp.int32) * V)[None, :]).reshape(B * F)
    emb2d = emb_table.reshape(F * V, D)
    lin1d = lin_w.reshape(F * V)
    linb16 = jnp.tile(lin_b.astype(jnp.float32), 16)
    out = _fm_sc()(flat_idx, emb2d, lin1d, linb16)
    return out.reshape(B, 1)


# speedup vs baseline: 1.9863x; 1.9863x over previous
import functools

import jax
import jax.numpy as jnp
from jax import lax
from jax.experimental import pallas as pl
from jax.experimental.pallas import tpu as pltpu
from jax.experimental.pallas import tpu_sc as plsc

B = 16384
F = 26
V = 100000
D = 16

NC = 2
NS = 16
NW = NC * NS
SPT = B // NW
C = 128
NCHUNK = SPT // C


def _fm_body(idxT_hbm, emb_hbm, lin_hbm, linb_hbm, out_hbm,
             idx_v, vals_v, lin_v, out_v, linb_v, sem):
    cid = lax.axis_index("c")
    sid = lax.axis_index("s")
    wid = sid * NC + cid

    pltpu.sync_copy(linb_hbm, linb_v)
    bvec = linb_v[...]

    def chunk_body(c, carry):
        base = wid * SPT + c * C
        for f in range(F):
            pltpu.sync_copy(idxT_hbm.at[f, pl.ds(base, C)], idx_v.at[f])
        copies = []
        for f in range(F):
            for d in range(D):
                copies.append(pltpu.async_copy(
                    emb_hbm.at[pl.ds((f * D + d) * V, V)].at[idx_v.at[f]],
                    vals_v.at[f * D + d], sem))
            copies.append(pltpu.async_copy(
                lin_hbm.at[pl.ds(f * V, V)].at[idx_v.at[f]], lin_v.at[f],
                sem))
        for cp in copies:
            cp.wait()

        def group_body(g, carry2):
            sl = pl.ds(g * 16, 16)
            acc = bvec
            lt = lin_v[0, sl]
            for f in range(1, F):
                lt = lt + lin_v[f, sl]
            acc = acc + lt
            for d in range(D):
                s = vals_v[d, sl]
                ss = s * s
                for f in range(1, F):
                    v = vals_v[f * D + d, sl]
                    s = s + v
                    ss = ss + v * v
                acc = acc + 0.5 * (s * s - ss)
            out_v[sl] = 1.0 / (1.0 + jnp.exp(-acc))
            return carry2

        lax.fori_loop(0, C // 16, group_body, 0)
        pltpu.sync_copy(out_v, out_hbm.at[pl.ds(base, C)])
        return carry

    lax.fori_loop(0, NCHUNK, chunk_body, 0)


@functools.cache
def _fm_sc():
    return pl.kernel(
        _fm_body,
        out_type=jax.ShapeDtypeStruct((B,), jnp.float32),
        mesh=plsc.VectorSubcoreMesh(
            core_axis_name="c", subcore_axis_name="s",
            num_cores=NC, num_subcores=NS),
        compiler_params=pltpu.CompilerParams(
            needs_layout_passes=False, use_tc_tiling_on_sc=False),
        scratch_types=[
            pltpu.VMEM((F, C), jnp.int32),
            pltpu.VMEM((F * D, C), jnp.float32),
            pltpu.VMEM((F, C), jnp.float32),
            pltpu.VMEM((C,), jnp.float32),
            pltpu.VMEM((16,), jnp.float32),
            pltpu.SemaphoreType.DMA,
        ],
    )


def kernel(indices, emb_table, lin_w, lin_b):
    idxT = indices.T.astype(jnp.int32)
    emb_flat = jnp.transpose(emb_table, (0, 2, 1)).reshape(F * D * V)
    lin_flat = lin_w.reshape(F * V)
    linb16 = jnp.tile(lin_b.astype(jnp.float32), 16)
    out = _fm_sc()(idxT, emb_flat, lin_flat, linb16)
    return out.reshape(B, 1)


# speedup vs baseline: 2.0739x; 1.0441x over previous
import functools

import jax
import jax.numpy as jnp
from jax import lax
from jax.experimental import pallas as pl
from jax.experimental.pallas import tpu as pltpu
from jax.experimental.pallas import tpu_sc as plsc

B = 16384
F = 26
V = 100000
D = 16

NC = 2
NS = 16
NW = NC * NS
SPT = B // NW
C = 256
NCHUNK = SPT // C


def _fm_body(idxT_hbm, emb_hbm, lin_hbm, linb_hbm, out_hbm,
             idx_v, vals_v, lin_v, out_v, linb_v, sem):
    cid = lax.axis_index("c")
    sid = lax.axis_index("s")
    wid = sid * NC + cid

    pltpu.sync_copy(linb_hbm, linb_v)
    bvec = linb_v[...]

    def chunk_body(c, carry):
        base = wid * SPT + c * C
        for f in range(F):
            pltpu.sync_copy(idxT_hbm.at[f, pl.ds(base, C)], idx_v.at[f])
        copies = []
        for f in range(F):
            for d in range(D):
                copies.append(pltpu.async_copy(
                    emb_hbm.at[pl.ds((f * D + d) * V, V)].at[idx_v.at[f]],
                    vals_v.at[f * D + d], sem))
            copies.append(pltpu.async_copy(
                lin_hbm.at[pl.ds(f * V, V)].at[idx_v.at[f]], lin_v.at[f],
                sem))
        for cp in copies:
            cp.wait()

        def group_body(g, carry2):
            sl = pl.ds(g * 16, 16)
            acc = bvec
            lt = lin_v[0, sl]
            for f in range(1, F):
                lt = lt + lin_v[f, sl]
            acc = acc + lt
            for d in range(D):
                s = vals_v[d, sl]
                ss = s * s
                for f in range(1, F):
                    v = vals_v[f * D + d, sl]
                    s = s + v
                    ss = ss + v * v
                acc = acc + 0.5 * (s * s - ss)
            out_v[sl] = 1.0 / (1.0 + jnp.exp(-acc))
            return carry2

        lax.fori_loop(0, C // 16, group_body, 0)
        pltpu.sync_copy(out_v, out_hbm.at[pl.ds(base, C)])
        return carry

    lax.fori_loop(0, NCHUNK, chunk_body, 0)


@functools.cache
def _fm_sc():
    return pl.kernel(
        _fm_body,
        out_type=jax.ShapeDtypeStruct((B,), jnp.float32),
        mesh=plsc.VectorSubcoreMesh(
            core_axis_name="c", subcore_axis_name="s",
            num_cores=NC, num_subcores=NS),
        compiler_params=pltpu.CompilerParams(
            needs_layout_passes=False, use_tc_tiling_on_sc=False),
        scratch_types=[
            pltpu.VMEM((F, C), jnp.int32),
            pltpu.VMEM((F * D, C), jnp.float32),
            pltpu.VMEM((F, C), jnp.float32),
            pltpu.VMEM((C,), jnp.float32),
            pltpu.VMEM((16,), jnp.float32),
            pltpu.SemaphoreType.DMA,
        ],
    )


def kernel(indices, emb_table, lin_w, lin_b):
    idxT = indices.T.astype(jnp.int32)
    emb_flat = jnp.transpose(emb_table, (0, 2, 1)).reshape(F * D * V)
    lin_flat = lin_w.reshape(F * V)
    linb16 = jnp.tile(lin_b.astype(jnp.float32), 16)
    out = _fm_sc()(idxT, emb_flat, lin_flat, linb16)
    return out.reshape(B, 1)
